# baseline (speedup 1.0000x reference)
import functools

import jax
import jax.numpy as jnp
from jax import lax
from jax.experimental import pallas as pl
from jax.experimental.pallas import tpu as pltpu
from jax.experimental.pallas import tpu_sc as plsc

_N_NODES = 10000
_N_EDGES = 320000
_D_FEAT = 128
_D_EDGE = 16
_D_OUT = 128


_NODE_BLK = 2000


def _tables_body(nodes_ref, wr_ref, ws_ref, nr_ref, ns_ref):
    n = nodes_ref[...]
    nr_ref[...] = jnp.dot(n, wr_ref[...], preferred_element_type=jnp.float32)
    ns_ref[...] = jnp.dot(n, ws_ref[...], preferred_element_type=jnp.float32)


def _make_tables(nodes, wr, ws):
    grid = _N_NODES // _NODE_BLK
    return pl.pallas_call(
        _tables_body,
        grid=(grid,),
        in_specs=[
            pl.BlockSpec((_NODE_BLK, _D_FEAT), lambda i: (i, 0)),
            pl.BlockSpec((_D_FEAT, _D_OUT), lambda i: (0, 0)),
            pl.BlockSpec((_D_FEAT, _D_OUT), lambda i: (0, 0)),
        ],
        out_specs=[
            pl.BlockSpec((_NODE_BLK, _D_OUT), lambda i: (i, 0)),
            pl.BlockSpec((_NODE_BLK, _D_OUT), lambda i: (i, 0)),
        ],
        out_shape=[
            jax.ShapeDtypeStruct((_N_NODES, _D_OUT), jnp.float32),
            jax.ShapeDtypeStruct((_N_NODES, _D_OUT), jnp.float32),
        ],
    )(nodes, wr, ws)


_EDGE_BLK = 3200


def _p_body(edges_t_ref, we_ref, g_ref, wg_ref, b_ref, p_ref):
    c = jnp.dot(g_ref[...], wg_ref[...], preferred_element_type=jnp.float32)
    c = c + b_ref[...]
    p = lax.dot_general(
        edges_t_ref[...], we_ref[...],
        (((0,), (0,)), ((), ())),
        preferred_element_type=jnp.float32,
    )
    p_ref[...] = p + c


def _make_p(edges_t, we, g, wg, b2d):
    grid = _N_EDGES // _EDGE_BLK
    return pl.pallas_call(
        _p_body,
        grid=(grid,),
        in_specs=[
            pl.BlockSpec((_D_EDGE, _EDGE_BLK), lambda i: (0, i)),
            pl.BlockSpec((_D_EDGE, _D_OUT), lambda i: (0, 0)),
            pl.BlockSpec((1, _D_FEAT), lambda i: (0, 0)),
            pl.BlockSpec((_D_FEAT, _D_OUT), lambda i: (0, 0)),
            pl.BlockSpec((1, _D_OUT), lambda i: (0, 0)),
        ],
        out_specs=pl.BlockSpec((_EDGE_BLK, _D_OUT), lambda i: (i, 0)),
        out_shape=jax.ShapeDtypeStruct((_N_EDGES, _D_OUT), jnp.float32),
    )(edges_t, we, g, wg, b2d)


_info = plsc.get_sparse_core_info()
_NC = _info.num_cores
_NS = _info.num_subcores
_NW = _NC * _NS
_PER_W = _N_EDGES // _NW
_CH = 80
_N_IT = _PER_W // _CH
assert _N_IT % 6 == 5, "pipeline below assumes N_IT = 6k+5"


@functools.partial(
    pl.kernel,
    mesh=plsc.VectorSubcoreMesh(core_axis_name="c", subcore_axis_name="s"),
    out_type=jax.ShapeDtypeStruct((_N_EDGES, _D_OUT), jnp.float32),
    scratch_types=[
        pltpu.VMEM((_PER_W,), jnp.int32),
        pltpu.VMEM((_PER_W,), jnp.int32),
        pltpu.VMEM((_CH, _D_OUT), jnp.float32),
        pltpu.VMEM((_CH, _D_OUT), jnp.float32),
        pltpu.VMEM((_CH, _D_OUT), jnp.float32),
        pltpu.VMEM((_CH, _D_OUT), jnp.float32),
        pltpu.VMEM((_CH, _D_OUT), jnp.float32),
        pltpu.VMEM((_CH, _D_OUT), jnp.float32),
        pltpu.VMEM((_CH, _D_OUT), jnp.float32),
        pltpu.VMEM((_CH, _D_OUT), jnp.float32),
        pltpu.VMEM((_CH, _D_OUT), jnp.float32),
        pltpu.VMEM((_CH, _D_OUT), jnp.float32),
        pltpu.SemaphoreType.DMA,
        pltpu.SemaphoreType.DMA,
        pltpu.SemaphoreType.DMA,
        pltpu.SemaphoreType.DMA,
        pltpu.SemaphoreType.DMA,
        pltpu.SemaphoreType.DMA,
        pltpu.SemaphoreType.DMA,
    ],
)
def _sc_edge(p_hbm, nr_hbm, ns_hbm, recv_hbm, send_hbm, out_hbm,
             idx_r_all, idx_s_all,
             br_0, bs_0, br_1, bs_1, br_2, bs_2,
             bp_0, bo_0, bp_1, bo_1,
             sem_0, sem_1, sem_2, psem_0, psem_1, sst_0, sst_1):
    wid = lax.axis_index("s") * _NC + lax.axis_index("c")
    w_base = pl.multiple_of(wid * _PER_W, 8)

    pltpu.sync_copy(recv_hbm.at[pl.ds(w_base, _PER_W)], idx_r_all)
    pltpu.sync_copy(send_hbm.at[pl.ds(w_base, _PER_W)], idx_s_all)

    gsets = ((br_0, bs_0, sem_0), (br_1, bs_1, sem_1), (br_2, bs_2, sem_2))
    qsets = ((bp_0, bo_0, psem_0, sst_0), (bp_1, bo_1, psem_1, sst_1))

    def fire_g(it, g):
        br, bs, sem = g
        off = pl.multiple_of(it * _CH, 8)
        pltpu.async_copy(nr_hbm.at[idx_r_all.at[pl.ds(off, _CH)]], br, sem)
        pltpu.async_copy(ns_hbm.at[idx_s_all.at[pl.ds(off, _CH)]], bs, sem)

    def fire_p(it, q):
        bp, bo, psem, sst = q
        base = w_base + pl.multiple_of(it * _CH, 8)
        pltpu.async_copy(p_hbm.at[pl.ds(base, _CH)], bp, psem)

    def step(c, g, q, do_g, do_p):
        br, bs, sem = g
        bp, bo, psem, sst = q
        off = pl.multiple_of(c * _CH, 8)
        base = w_base + off
        pltpu.make_async_copy(nr_hbm.at[idx_r_all.at[pl.ds(off, _CH)]], br, sem).wait()
        pltpu.make_async_copy(ns_hbm.at[idx_s_all.at[pl.ds(off, _CH)]], bs, sem).wait()
        pltpu.make_async_copy(p_hbm.at[pl.ds(base, _CH)], bp, psem).wait()

        @pl.when(c >= 2)
        def _():
            pltpu.make_async_copy(bo, out_hbm.at[pl.ds(w_base, _CH)], sst).wait()

        def row(r, acc):
            for cg in range(_D_OUT // 16):
                sl = pl.ds(cg * 16, 16)
                bo[r, sl] = jnp.maximum(bp[r, sl] + br[r, sl] + bs[r, sl], 0.0)
            return acc

        lax.fori_loop(0, _CH, row, 0)
        pltpu.async_copy(bo, out_hbm.at[pl.ds(base, _CH)], sst)

        if do_g:
            fire_g(c + 3, g)
        if do_p:
            fire_p(c + 2, q)

    fire_g(0, gsets[0])
    fire_g(1, gsets[1])
    fire_g(2, gsets[2])
    fire_p(0, qsets[0])
    fire_p(1, qsets[1])

    def body(j, carry):
        for t in range(6):
            step(6 * j + t, gsets[t % 3], qsets[t % 2], True, True)
        return carry

    lax.fori_loop(0, (_N_IT - 5) // 6, body, 0)
    for c in range(_N_IT - 5, _N_IT):
        step(c, gsets[c % 3], qsets[c % 2], c + 3 < _N_IT, c + 2 < _N_IT)
    pltpu.make_async_copy(bo_0, out_hbm.at[pl.ds(w_base, _CH)], sst_0).wait()
    pltpu.make_async_copy(bo_1, out_hbm.at[pl.ds(w_base, _CH)], sst_1).wait()


def kernel(nodes, edges, globals_attr, senders, receivers, W, b):
    we = W[:_D_EDGE]
    wr = W[_D_EDGE:_D_EDGE + _D_FEAT]
    ws = W[_D_EDGE + _D_FEAT:_D_EDGE + 2 * _D_FEAT]
    wg = W[_D_EDGE + 2 * _D_FEAT:]
    nr, ns = _make_tables(nodes, wr, ws)
    p = _make_p(edges.T, we, globals_attr, wg, b.reshape(1, _D_OUT))
    return _sc_edge(p, nr, ns, receivers, senders)


# speedup vs baseline: 1.1328x; 1.1328x over previous
import functools

import jax
import jax.numpy as jnp
from jax import lax
from jax.experimental import pallas as pl
from jax.experimental.pallas import tpu as pltpu
from jax.experimental.pallas import tpu_sc as plsc

_N_NODES = 10000
_N_EDGES = 320000
_D_FEAT = 128
_D_EDGE = 16
_D_OUT = 128


_NODE_BLK = 2000


def _tables_body(nodes_ref, wr_ref, ws_ref, nr_ref, ns_ref):
    n = nodes_ref[...]
    nr_ref[...] = jnp.dot(n, wr_ref[...], preferred_element_type=jnp.float32)
    ns_ref[...] = jnp.dot(n, ws_ref[...], preferred_element_type=jnp.float32)


def _make_tables(nodes, wr, ws):
    grid = _N_NODES // _NODE_BLK
    return pl.pallas_call(
        _tables_body,
        grid=(grid,),
        in_specs=[
            pl.BlockSpec((_NODE_BLK, _D_FEAT), lambda i: (i, 0)),
            pl.BlockSpec((_D_FEAT, _D_OUT), lambda i: (0, 0)),
            pl.BlockSpec((_D_FEAT, _D_OUT), lambda i: (0, 0)),
        ],
        out_specs=[
            pl.BlockSpec((_NODE_BLK, _D_OUT), lambda i: (i, 0)),
            pl.BlockSpec((_NODE_BLK, _D_OUT), lambda i: (i, 0)),
        ],
        out_shape=[
            jax.ShapeDtypeStruct((_N_NODES, _D_OUT), jnp.float32),
            jax.ShapeDtypeStruct((_N_NODES, _D_OUT), jnp.float32),
        ],
    )(nodes, wr, ws)


_EDGE_BLK = 12800


def _p_body(edges_t_ref, we_ref, g_ref, wg_ref, b_ref, p_ref):
    c = jnp.dot(g_ref[...], wg_ref[...], preferred_element_type=jnp.float32)
    c = c + b_ref[...]
    p = lax.dot_general(
        edges_t_ref[...], we_ref[...],
        (((0,), (0,)), ((), ())),
        preferred_element_type=jnp.float32,
    )
    p_ref[...] = p + c


def _make_p(edges_t, we, g, wg, b2d):
    grid = _N_EDGES // _EDGE_BLK
    return pl.pallas_call(
        _p_body,
        grid=(grid,),
        in_specs=[
            pl.BlockSpec((_D_EDGE, _EDGE_BLK), lambda i: (0, i)),
            pl.BlockSpec((_D_EDGE, _D_OUT), lambda i: (0, 0)),
            pl.BlockSpec((1, _D_FEAT), lambda i: (0, 0)),
            pl.BlockSpec((_D_FEAT, _D_OUT), lambda i: (0, 0)),
            pl.BlockSpec((1, _D_OUT), lambda i: (0, 0)),
        ],
        out_specs=pl.BlockSpec((_EDGE_BLK, _D_OUT), lambda i: (i, 0)),
        out_shape=jax.ShapeDtypeStruct((_N_EDGES, _D_OUT), jnp.float32),
    )(edges_t, we, g, wg, b2d)


_info = plsc.get_sparse_core_info()
_NC = _info.num_cores
_NS = _info.num_subcores
_NW = _NC * _NS
_PER_W = _N_EDGES // _NW
_CH = 80
_N_IT = _PER_W // _CH
assert _N_IT % 6 == 5, "pipeline below assumes N_IT = 6k+5"


@functools.partial(
    pl.kernel,
    mesh=plsc.VectorSubcoreMesh(core_axis_name="c", subcore_axis_name="s"),
    out_type=jax.ShapeDtypeStruct((_N_EDGES, _D_OUT), jnp.float32),
    scratch_types=[
        pltpu.VMEM((_PER_W,), jnp.int32),
        pltpu.VMEM((_PER_W,), jnp.int32),
        pltpu.VMEM((_CH, _D_OUT), jnp.float32),
        pltpu.VMEM((_CH, _D_OUT), jnp.float32),
        pltpu.VMEM((_CH, _D_OUT), jnp.float32),
        pltpu.VMEM((_CH, _D_OUT), jnp.float32),
        pltpu.VMEM((_CH, _D_OUT), jnp.float32),
        pltpu.VMEM((_CH, _D_OUT), jnp.float32),
        pltpu.VMEM((_CH, _D_OUT), jnp.float32),
        pltpu.VMEM((_CH, _D_OUT), jnp.float32),
        pltpu.VMEM((_CH, _D_OUT), jnp.float32),
        pltpu.VMEM((_CH, _D_OUT), jnp.float32),
        pltpu.SemaphoreType.DMA,
        pltpu.SemaphoreType.DMA,
        pltpu.SemaphoreType.DMA,
        pltpu.SemaphoreType.DMA,
        pltpu.SemaphoreType.DMA,
        pltpu.SemaphoreType.DMA,
        pltpu.SemaphoreType.DMA,
    ],
)
def _sc_edge(p_hbm, nr_hbm, ns_hbm, recv_hbm, send_hbm, out_hbm,
             idx_r_all, idx_s_all,
             br_0, bs_0, br_1, bs_1, br_2, bs_2,
             bp_0, bo_0, bp_1, bo_1,
             sem_0, sem_1, sem_2, psem_0, psem_1, sst_0, sst_1):
    wid = lax.axis_index("s") * _NC + lax.axis_index("c")
    w_base = pl.multiple_of(wid * _PER_W, 8)

    pltpu.sync_copy(recv_hbm.at[pl.ds(w_base, _PER_W)], idx_r_all)
    pltpu.sync_copy(send_hbm.at[pl.ds(w_base, _PER_W)], idx_s_all)

    gsets = ((br_0, bs_0, sem_0), (br_1, bs_1, sem_1), (br_2, bs_2, sem_2))
    qsets = ((bp_0, bo_0, psem_0, sst_0), (bp_1, bo_1, psem_1, sst_1))

    def fire_g(it, g):
        br, bs, sem = g
        off = pl.multiple_of(it * _CH, 8)
        pltpu.async_copy(nr_hbm.at[idx_r_all.at[pl.ds(off, _CH)]], br, sem)
        pltpu.async_copy(ns_hbm.at[idx_s_all.at[pl.ds(off, _CH)]], bs, sem)

    def fire_p(it, q):
        bp, bo, psem, sst = q
        base = w_base + pl.multiple_of(it * _CH, 8)
        pltpu.async_copy(p_hbm.at[pl.ds(base, _CH)], bp, psem)

    def step(c, g, q, do_g, do_p):
        br, bs, sem = g
        bp, bo, psem, sst = q
        off = pl.multiple_of(c * _CH, 8)
        base = w_base + off
        pltpu.make_async_copy(nr_hbm.at[idx_r_all.at[pl.ds(off, _CH)]], br, sem).wait()
        pltpu.make_async_copy(ns_hbm.at[idx_s_all.at[pl.ds(off, _CH)]], bs, sem).wait()
        pltpu.make_async_copy(p_hbm.at[pl.ds(base, _CH)], bp, psem).wait()

        @pl.when(c >= 2)
        def _():
            pltpu.make_async_copy(bo, out_hbm.at[pl.ds(w_base, _CH)], sst).wait()

        def row(r, acc):
            for cg in range(_D_OUT // 16):
                sl = pl.ds(cg * 16, 16)
                bo[r, sl] = jnp.maximum(bp[r, sl] + br[r, sl] + bs[r, sl], 0.0)
            return acc

        lax.fori_loop(0, _CH, row, 0)
        pltpu.async_copy(bo, out_hbm.at[pl.ds(base, _CH)], sst)

        if do_g:
            fire_g(c + 3, g)
        if do_p:
            fire_p(c + 2, q)

    fire_g(0, gsets[0])
    fire_g(1, gsets[1])
    fire_g(2, gsets[2])
    fire_p(0, qsets[0])
    fire_p(1, qsets[1])

    def body(j, carry):
        for t in range(6):
            step(6 * j + t, gsets[t % 3], qsets[t % 2], True, True)
        return carry

    lax.fori_loop(0, (_N_IT - 5) // 6, body, 0)
    for c in range(_N_IT - 5, _N_IT):
        step(c, gsets[c % 3], qsets[c % 2], c + 3 < _N_IT, c + 2 < _N_IT)
    pltpu.make_async_copy(bo_0, out_hbm.at[pl.ds(w_base, _CH)], sst_0).wait()
    pltpu.make_async_copy(bo_1, out_hbm.at[pl.ds(w_base, _CH)], sst_1).wait()


def kernel(nodes, edges, globals_attr, senders, receivers, W, b):
    we = W[:_D_EDGE]
    wr = W[_D_EDGE:_D_EDGE + _D_FEAT]
    ws = W[_D_EDGE + _D_FEAT:_D_EDGE + 2 * _D_FEAT]
    wg = W[_D_EDGE + 2 * _D_FEAT:]
    nr, ns = _make_tables(nodes, wr, ws)
    p = _make_p(edges.T, we, globals_attr, wg, b.reshape(1, _D_OUT))
    return _sc_edge(p, nr, ns, receivers, senders)


# speedup vs baseline: 1.1432x; 1.0091x over previous
import functools

import jax
import jax.numpy as jnp
from jax import lax
from jax.experimental import pallas as pl
from jax.experimental.pallas import tpu as pltpu
from jax.experimental.pallas import tpu_sc as plsc

_N_NODES = 10000
_N_EDGES = 320000
_D_FEAT = 128
_D_EDGE = 16
_D_OUT = 128


_NODE_BLK = 2000


def _tables_body(nodes_ref, wr_ref, ws_ref, nr_ref, ns_ref):
    n = nodes_ref[...]
    nr_ref[...] = jnp.dot(n, wr_ref[...], preferred_element_type=jnp.float32)
    ns_ref[...] = jnp.dot(n, ws_ref[...], preferred_element_type=jnp.float32)


def _make_tables(nodes, wr, ws):
    grid = _N_NODES // _NODE_BLK
    return pl.pallas_call(
        _tables_body,
        grid=(grid,),
        in_specs=[
            pl.BlockSpec((_NODE_BLK, _D_FEAT), lambda i: (i, 0)),
            pl.BlockSpec((_D_FEAT, _D_OUT), lambda i: (0, 0)),
            pl.BlockSpec((_D_FEAT, _D_OUT), lambda i: (0, 0)),
        ],
        out_specs=[
            pl.BlockSpec((_NODE_BLK, _D_OUT), lambda i: (i, 0)),
            pl.BlockSpec((_NODE_BLK, _D_OUT), lambda i: (i, 0)),
        ],
        out_shape=[
            jax.ShapeDtypeStruct((_N_NODES, _D_OUT), jnp.float32),
            jax.ShapeDtypeStruct((_N_NODES, _D_OUT), jnp.float32),
        ],
    )(nodes, wr, ws)


_EDGE_BLK = 16000


def _p_body(edges_t_ref, we_ref, g_ref, wg_ref, b_ref, p_ref):
    c = jnp.dot(g_ref[...], wg_ref[...], preferred_element_type=jnp.float32)
    c = c + b_ref[...]
    p = lax.dot_general(
        edges_t_ref[...], we_ref[...],
        (((0,), (0,)), ((), ())),
        preferred_element_type=jnp.float32,
    )
    p_ref[...] = p + c


def _make_p(edges_t, we, g, wg, b2d):
    grid = _N_EDGES // _EDGE_BLK
    return pl.pallas_call(
        _p_body,
        grid=(grid,),
        in_specs=[
            pl.BlockSpec((_D_EDGE, _EDGE_BLK), lambda i: (0, i)),
            pl.BlockSpec((_D_EDGE, _D_OUT), lambda i: (0, 0)),
            pl.BlockSpec((1, _D_FEAT), lambda i: (0, 0)),
            pl.BlockSpec((_D_FEAT, _D_OUT), lambda i: (0, 0)),
            pl.BlockSpec((1, _D_OUT), lambda i: (0, 0)),
        ],
        out_specs=pl.BlockSpec((_EDGE_BLK, _D_OUT), lambda i: (i, 0)),
        out_shape=jax.ShapeDtypeStruct((_N_EDGES, _D_OUT), jnp.float32),
    )(edges_t, we, g, wg, b2d)


_info = plsc.get_sparse_core_info()
_NC = _info.num_cores
_NS = _info.num_subcores
_NW = _NC * _NS
_PER_W = _N_EDGES // _NW
_CH = 80
_N_IT = _PER_W // _CH
assert _N_IT % 6 == 5, "pipeline below assumes N_IT = 6k+5"


@functools.partial(
    pl.kernel,
    mesh=plsc.VectorSubcoreMesh(core_axis_name="c", subcore_axis_name="s"),
    out_type=jax.ShapeDtypeStruct((_N_EDGES, _D_OUT), jnp.float32),
    scratch_types=[
        pltpu.VMEM((_PER_W,), jnp.int32),
        pltpu.VMEM((_PER_W,), jnp.int32),
        pltpu.VMEM((_CH, _D_OUT), jnp.float32),
        pltpu.VMEM((_CH, _D_OUT), jnp.float32),
        pltpu.VMEM((_CH, _D_OUT), jnp.float32),
        pltpu.VMEM((_CH, _D_OUT), jnp.float32),
        pltpu.VMEM((_CH, _D_OUT), jnp.float32),
        pltpu.VMEM((_CH, _D_OUT), jnp.float32),
        pltpu.VMEM((_CH, _D_OUT), jnp.float32),
        pltpu.VMEM((_CH, _D_OUT), jnp.float32),
        pltpu.VMEM((_CH, _D_OUT), jnp.float32),
        pltpu.VMEM((_CH, _D_OUT), jnp.float32),
        pltpu.SemaphoreType.DMA,
        pltpu.SemaphoreType.DMA,
        pltpu.SemaphoreType.DMA,
        pltpu.SemaphoreType.DMA,
        pltpu.SemaphoreType.DMA,
        pltpu.SemaphoreType.DMA,
        pltpu.SemaphoreType.DMA,
    ],
)
def _sc_edge(p_hbm, nr_hbm, ns_hbm, recv_hbm, send_hbm, out_hbm,
             idx_r_all, idx_s_all,
             br_0, bs_0, br_1, bs_1, br_2, bs_2,
             bp_0, bo_0, bp_1, bo_1,
             sem_0, sem_1, sem_2, psem_0, psem_1, sst_0, sst_1):
    wid = lax.axis_index("s") * _NC + lax.axis_index("c")
    w_base = pl.multiple_of(wid * _PER_W, 8)

    pltpu.sync_copy(recv_hbm.at[pl.ds(w_base, _PER_W)], idx_r_all)
    pltpu.sync_copy(send_hbm.at[pl.ds(w_base, _PER_W)], idx_s_all)

    gsets = ((br_0, bs_0, sem_0), (br_1, bs_1, sem_1), (br_2, bs_2, sem_2))
    qsets = ((bp_0, bo_0, psem_0, sst_0), (bp_1, bo_1, psem_1, sst_1))

    def fire_g(it, g):
        br, bs, sem = g
        off = pl.multiple_of(it * _CH, 8)
        pltpu.async_copy(nr_hbm.at[idx_r_all.at[pl.ds(off, _CH)]], br, sem)
        pltpu.async_copy(ns_hbm.at[idx_s_all.at[pl.ds(off, _CH)]], bs, sem)

    def fire_p(it, q):
        bp, bo, psem, sst = q
        base = w_base + pl.multiple_of(it * _CH, 8)
        pltpu.async_copy(p_hbm.at[pl.ds(base, _CH)], bp, psem)

    def step(c, g, q, do_g, do_p):
        br, bs, sem = g
        bp, bo, psem, sst = q
        off = pl.multiple_of(c * _CH, 8)
        base = w_base + off
        pltpu.make_async_copy(nr_hbm.at[idx_r_all.at[pl.ds(off, _CH)]], br, sem).wait()
        pltpu.make_async_copy(ns_hbm.at[idx_s_all.at[pl.ds(off, _CH)]], bs, sem).wait()
        pltpu.make_async_copy(p_hbm.at[pl.ds(base, _CH)], bp, psem).wait()

        @pl.when(c >= 2)
        def _():
            pltpu.make_async_copy(bo, out_hbm.at[pl.ds(w_base, _CH)], sst).wait()

        def row(r, acc):
            for cg in range(_D_OUT // 16):
                sl = pl.ds(cg * 16, 16)
                bo[r, sl] = jnp.maximum(bp[r, sl] + br[r, sl] + bs[r, sl], 0.0)
            return acc

        lax.fori_loop(0, _CH, row, 0)
        pltpu.async_copy(bo, out_hbm.at[pl.ds(base, _CH)], sst)

        if do_g:
            fire_g(c + 3, g)
        if do_p:
            fire_p(c + 2, q)

    fire_g(0, gsets[0])
    fire_g(1, gsets[1])
    fire_g(2, gsets[2])
    fire_p(0, qsets[0])
    fire_p(1, qsets[1])

    def body(j, carry):
        for t in range(6):
            step(6 * j + t, gsets[t % 3], qsets[t % 2], True, True)
        return carry

    lax.fori_loop(0, (_N_IT - 5) // 6, body, 0)
    for c in range(_N_IT - 5, _N_IT):
        step(c, gsets[c % 3], qsets[c % 2], c + 3 < _N_IT, c + 2 < _N_IT)
    pltpu.make_async_copy(bo_0, out_hbm.at[pl.ds(w_base, _CH)], sst_0).wait()
    pltpu.make_async_copy(bo_1, out_hbm.at[pl.ds(w_base, _CH)], sst_1).wait()


def kernel(nodes, edges, globals_attr, senders, receivers, W, b):
    we = W[:_D_EDGE]
    wr = W[_D_EDGE:_D_EDGE + _D_FEAT]
    ws = W[_D_EDGE + _D_FEAT:_D_EDGE + 2 * _D_FEAT]
    wg = W[_D_EDGE + 2 * _D_FEAT:]
    nr, ns = _make_tables(nodes, wr, ws)
    p = _make_p(edges.T, we, globals_attr, wg, b.reshape(1, _D_OUT))
    return _sc_edge(p, nr, ns, receivers, senders)


# speedup vs baseline: 1.1461x; 1.0026x over previous
import functools

import jax
import jax.numpy as jnp
from jax import lax
from jax.experimental import pallas as pl
from jax.experimental.pallas import tpu as pltpu
from jax.experimental.pallas import tpu_sc as plsc

_N_NODES = 10000
_N_EDGES = 320000
_D_FEAT = 128
_D_EDGE = 16
_D_OUT = 128


_NODE_BLK = 2000


def _tables_body(nodes_ref, wr_ref, ws_ref, nr_ref, ns_ref):
    n = nodes_ref[...]
    nr_ref[...] = jnp.dot(n, wr_ref[...], preferred_element_type=jnp.float32)
    ns_ref[...] = jnp.dot(n, ws_ref[...], preferred_element_type=jnp.float32)


def _make_tables(nodes, wr, ws):
    grid = _N_NODES // _NODE_BLK
    return pl.pallas_call(
        _tables_body,
        grid=(grid,),
        in_specs=[
            pl.BlockSpec((_NODE_BLK, _D_FEAT), lambda i: (i, 0)),
            pl.BlockSpec((_D_FEAT, _D_OUT), lambda i: (0, 0)),
            pl.BlockSpec((_D_FEAT, _D_OUT), lambda i: (0, 0)),
        ],
        out_specs=[
            pl.BlockSpec((_NODE_BLK, _D_OUT), lambda i: (i, 0)),
            pl.BlockSpec((_NODE_BLK, _D_OUT), lambda i: (i, 0)),
        ],
        out_shape=[
            jax.ShapeDtypeStruct((_N_NODES, _D_OUT), jnp.float32),
            jax.ShapeDtypeStruct((_N_NODES, _D_OUT), jnp.float32),
        ],
    )(nodes, wr, ws)


_EDGE_BLK = 32000


def _p_body(edges_t_ref, we_ref, g_ref, wg_ref, b_ref, p_ref):
    c = jnp.dot(g_ref[...], wg_ref[...], preferred_element_type=jnp.float32)
    c = c + b_ref[...]
    p = lax.dot_general(
        edges_t_ref[...], we_ref[...],
        (((0,), (0,)), ((), ())),
        preferred_element_type=jnp.float32,
    )
    p_ref[...] = p + c


def _make_p(edges_t, we, g, wg, b2d):
    grid = _N_EDGES // _EDGE_BLK
    return pl.pallas_call(
        _p_body,
        grid=(grid,),
        in_specs=[
            pl.BlockSpec((_D_EDGE, _EDGE_BLK), lambda i: (0, i)),
            pl.BlockSpec((_D_EDGE, _D_OUT), lambda i: (0, 0)),
            pl.BlockSpec((1, _D_FEAT), lambda i: (0, 0)),
            pl.BlockSpec((_D_FEAT, _D_OUT), lambda i: (0, 0)),
            pl.BlockSpec((1, _D_OUT), lambda i: (0, 0)),
        ],
        out_specs=pl.BlockSpec((_EDGE_BLK, _D_OUT), lambda i: (i, 0)),
        out_shape=jax.ShapeDtypeStruct((_N_EDGES, _D_OUT), jnp.float32),
    )(edges_t, we, g, wg, b2d)


_info = plsc.get_sparse_core_info()
_NC = _info.num_cores
_NS = _info.num_subcores
_NW = _NC * _NS
_PER_W = _N_EDGES // _NW
_CH = 80
_N_IT = _PER_W // _CH
assert _N_IT % 6 == 5, "pipeline below assumes N_IT = 6k+5"


@functools.partial(
    pl.kernel,
    mesh=plsc.VectorSubcoreMesh(core_axis_name="c", subcore_axis_name="s"),
    out_type=jax.ShapeDtypeStruct((_N_EDGES, _D_OUT), jnp.float32),
    scratch_types=[
        pltpu.VMEM((_PER_W,), jnp.int32),
        pltpu.VMEM((_PER_W,), jnp.int32),
        pltpu.VMEM((_CH, _D_OUT), jnp.float32),
        pltpu.VMEM((_CH, _D_OUT), jnp.float32),
        pltpu.VMEM((_CH, _D_OUT), jnp.float32),
        pltpu.VMEM((_CH, _D_OUT), jnp.float32),
        pltpu.VMEM((_CH, _D_OUT), jnp.float32),
        pltpu.VMEM((_CH, _D_OUT), jnp.float32),
        pltpu.VMEM((_CH, _D_OUT), jnp.float32),
        pltpu.VMEM((_CH, _D_OUT), jnp.float32),
        pltpu.VMEM((_CH, _D_OUT), jnp.float32),
        pltpu.VMEM((_CH, _D_OUT), jnp.float32),
        pltpu.SemaphoreType.DMA,
        pltpu.SemaphoreType.DMA,
        pltpu.SemaphoreType.DMA,
        pltpu.SemaphoreType.DMA,
        pltpu.SemaphoreType.DMA,
        pltpu.SemaphoreType.DMA,
        pltpu.SemaphoreType.DMA,
    ],
)
def _sc_edge(p_hbm, nr_hbm, ns_hbm, recv_hbm, send_hbm, out_hbm,
             idx_r_all, idx_s_all,
             br_0, bs_0, br_1, bs_1, br_2, bs_2,
             bp_0, bo_0, bp_1, bo_1,
             sem_0, sem_1, sem_2, psem_0, psem_1, sst_0, sst_1):
    wid = lax.axis_index("s") * _NC + lax.axis_index("c")
    w_base = pl.multiple_of(wid * _PER_W, 8)

    pltpu.sync_copy(recv_hbm.at[pl.ds(w_base, _PER_W)], idx_r_all)
    pltpu.sync_copy(send_hbm.at[pl.ds(w_base, _PER_W)], idx_s_all)

    gsets = ((br_0, bs_0, sem_0), (br_1, bs_1, sem_1), (br_2, bs_2, sem_2))
    qsets = ((bp_0, bo_0, psem_0, sst_0), (bp_1, bo_1, psem_1, sst_1))

    def fire_g(it, g):
        br, bs, sem = g
        off = pl.multiple_of(it * _CH, 8)
        pltpu.async_copy(nr_hbm.at[idx_r_all.at[pl.ds(off, _CH)]], br, sem)
        pltpu.async_copy(ns_hbm.at[idx_s_all.at[pl.ds(off, _CH)]], bs, sem)

    def fire_p(it, q):
        bp, bo, psem, sst = q
        base = w_base + pl.multiple_of(it * _CH, 8)
        pltpu.async_copy(p_hbm.at[pl.ds(base, _CH)], bp, psem)

    def step(c, g, q, do_g, do_p):
        br, bs, sem = g
        bp, bo, psem, sst = q
        off = pl.multiple_of(c * _CH, 8)
        base = w_base + off
        pltpu.make_async_copy(nr_hbm.at[idx_r_all.at[pl.ds(off, _CH)]], br, sem).wait()
        pltpu.make_async_copy(ns_hbm.at[idx_s_all.at[pl.ds(off, _CH)]], bs, sem).wait()
        pltpu.make_async_copy(p_hbm.at[pl.ds(base, _CH)], bp, psem).wait()

        @pl.when(c >= 2)
        def _():
            pltpu.make_async_copy(bo, out_hbm.at[pl.ds(w_base, _CH)], sst).wait()

        def row(r, acc):
            for cg in range(_D_OUT // 16):
                sl = pl.ds(cg * 16, 16)
                bo[r, sl] = jnp.maximum(bp[r, sl] + br[r, sl] + bs[r, sl], 0.0)
            return acc

        lax.fori_loop(0, _CH, row, 0)
        pltpu.async_copy(bo, out_hbm.at[pl.ds(base, _CH)], sst)

        if do_g:
            fire_g(c + 3, g)
        if do_p:
            fire_p(c + 2, q)

    fire_g(0, gsets[0])
    fire_g(1, gsets[1])
    fire_g(2, gsets[2])
    fire_p(0, qsets[0])
    fire_p(1, qsets[1])

    def body(j, carry):
        for t in range(6):
            step(6 * j + t, gsets[t % 3], qsets[t % 2], True, True)
        return carry

    lax.fori_loop(0, (_N_IT - 5) // 6, body, 0)
    for c in range(_N_IT - 5, _N_IT):
        step(c, gsets[c % 3], qsets[c % 2], c + 3 < _N_IT, c + 2 < _N_IT)
    pltpu.make_async_copy(bo_0, out_hbm.at[pl.ds(w_base, _CH)], sst_0).wait()
    pltpu.make_async_copy(bo_1, out_hbm.at[pl.ds(w_base, _CH)], sst_1).wait()


def kernel(nodes, edges, globals_attr, senders, receivers, W, b):
    we = W[:_D_EDGE]
    wr = W[_D_EDGE:_D_EDGE + _D_FEAT]
    ws = W[_D_EDGE + _D_FEAT:_D_EDGE + 2 * _D_FEAT]
    wg = W[_D_EDGE + 2 * _D_FEAT:]
    nr, ns = _make_tables(nodes, wr, ws)
    p = _make_p(edges.T, we, globals_attr, wg, b.reshape(1, _D_OUT))
    return _sc_edge(p, nr, ns, receivers, senders)


# speedup vs baseline: 1.1471x; 1.0008x over previous
import functools

import jax
import jax.numpy as jnp
from jax import lax
from jax.experimental import pallas as pl
from jax.experimental.pallas import tpu as pltpu
from jax.experimental.pallas import tpu_sc as plsc

_N_NODES = 10000
_N_EDGES = 320000
_D_FEAT = 128
_D_EDGE = 16
_D_OUT = 128


_NODE_BLK = 10000


def _tables_body(nodes_ref, wr_ref, ws_ref, nr_ref, ns_ref):
    n = nodes_ref[...]
    nr_ref[...] = jnp.dot(n, wr_ref[...], preferred_element_type=jnp.float32)
    ns_ref[...] = jnp.dot(n, ws_ref[...], preferred_element_type=jnp.float32)


def _make_tables(nodes, wr, ws):
    grid = _N_NODES // _NODE_BLK
    return pl.pallas_call(
        _tables_body,
        grid=(grid,),
        in_specs=[
            pl.BlockSpec((_NODE_BLK, _D_FEAT), lambda i: (i, 0)),
            pl.BlockSpec((_D_FEAT, _D_OUT), lambda i: (0, 0)),
            pl.BlockSpec((_D_FEAT, _D_OUT), lambda i: (0, 0)),
        ],
        out_specs=[
            pl.BlockSpec((_NODE_BLK, _D_OUT), lambda i: (i, 0)),
            pl.BlockSpec((_NODE_BLK, _D_OUT), lambda i: (i, 0)),
        ],
        out_shape=[
            jax.ShapeDtypeStruct((_N_NODES, _D_OUT), jnp.float32),
            jax.ShapeDtypeStruct((_N_NODES, _D_OUT), jnp.float32),
        ],
    )(nodes, wr, ws)


_EDGE_BLK = 32000


def _p_body(edges_t_ref, we_ref, g_ref, wg_ref, b_ref, p_ref):
    c = jnp.dot(g_ref[...], wg_ref[...], preferred_element_type=jnp.float32)
    c = c + b_ref[...]
    p = lax.dot_general(
        edges_t_ref[...], we_ref[...],
        (((0,), (0,)), ((), ())),
        preferred_element_type=jnp.float32,
    )
    p_ref[...] = p + c


def _make_p(edges_t, we, g, wg, b2d):
    grid = _N_EDGES // _EDGE_BLK
    return pl.pallas_call(
        _p_body,
        grid=(grid,),
        in_specs=[
            pl.BlockSpec((_D_EDGE, _EDGE_BLK), lambda i: (0, i)),
            pl.BlockSpec((_D_EDGE, _D_OUT), lambda i: (0, 0)),
            pl.BlockSpec((1, _D_FEAT), lambda i: (0, 0)),
            pl.BlockSpec((_D_FEAT, _D_OUT), lambda i: (0, 0)),
            pl.BlockSpec((1, _D_OUT), lambda i: (0, 0)),
        ],
        out_specs=pl.BlockSpec((_EDGE_BLK, _D_OUT), lambda i: (i, 0)),
        out_shape=jax.ShapeDtypeStruct((_N_EDGES, _D_OUT), jnp.float32),
    )(edges_t, we, g, wg, b2d)


_info = plsc.get_sparse_core_info()
_NC = _info.num_cores
_NS = _info.num_subcores
_NW = _NC * _NS
_PER_W = _N_EDGES // _NW
_CH = 80
_N_IT = _PER_W // _CH
assert _N_IT % 6 == 5, "pipeline below assumes N_IT = 6k+5"


@functools.partial(
    pl.kernel,
    mesh=plsc.VectorSubcoreMesh(core_axis_name="c", subcore_axis_name="s"),
    out_type=jax.ShapeDtypeStruct((_N_EDGES, _D_OUT), jnp.float32),
    scratch_types=[
        pltpu.VMEM((_PER_W,), jnp.int32),
        pltpu.VMEM((_PER_W,), jnp.int32),
        pltpu.VMEM((_CH, _D_OUT), jnp.float32),
        pltpu.VMEM((_CH, _D_OUT), jnp.float32),
        pltpu.VMEM((_CH, _D_OUT), jnp.float32),
        pltpu.VMEM((_CH, _D_OUT), jnp.float32),
        pltpu.VMEM((_CH, _D_OUT), jnp.float32),
        pltpu.VMEM((_CH, _D_OUT), jnp.float32),
        pltpu.VMEM((_CH, _D_OUT), jnp.float32),
        pltpu.VMEM((_CH, _D_OUT), jnp.float32),
        pltpu.VMEM((_CH, _D_OUT), jnp.float32),
        pltpu.VMEM((_CH, _D_OUT), jnp.float32),
        pltpu.SemaphoreType.DMA,
        pltpu.SemaphoreType.DMA,
        pltpu.SemaphoreType.DMA,
        pltpu.SemaphoreType.DMA,
        pltpu.SemaphoreType.DMA,
        pltpu.SemaphoreType.DMA,
        pltpu.SemaphoreType.DMA,
    ],
)
def _sc_edge(p_hbm, nr_hbm, ns_hbm, recv_hbm, send_hbm, out_hbm,
             idx_r_all, idx_s_all,
             br_0, bs_0, br_1, bs_1, br_2, bs_2,
             bp_0, bo_0, bp_1, bo_1,
             sem_0, sem_1, sem_2, psem_0, psem_1, sst_0, sst_1):
    wid = lax.axis_index("s") * _NC + lax.axis_index("c")
    w_base = pl.multiple_of(wid * _PER_W, 8)

    pltpu.sync_copy(recv_hbm.at[pl.ds(w_base, _PER_W)], idx_r_all)
    pltpu.sync_copy(send_hbm.at[pl.ds(w_base, _PER_W)], idx_s_all)

    gsets = ((br_0, bs_0, sem_0), (br_1, bs_1, sem_1), (br_2, bs_2, sem_2))
    qsets = ((bp_0, bo_0, psem_0, sst_0), (bp_1, bo_1, psem_1, sst_1))

    def fire_g(it, g):
        br, bs, sem = g
        off = pl.multiple_of(it * _CH, 8)
        pltpu.async_copy(nr_hbm.at[idx_r_all.at[pl.ds(off, _CH)]], br, sem)
        pltpu.async_copy(ns_hbm.at[idx_s_all.at[pl.ds(off, _CH)]], bs, sem)

    def fire_p(it, q):
        bp, bo, psem, sst = q
        base = w_base + pl.multiple_of(it * _CH, 8)
        pltpu.async_copy(p_hbm.at[pl.ds(base, _CH)], bp, psem)

    def step(c, g, q, do_g, do_p):
        br, bs, sem = g
        bp, bo, psem, sst = q
        off = pl.multiple_of(c * _CH, 8)
        base = w_base + off
        pltpu.make_async_copy(nr_hbm.at[idx_r_all.at[pl.ds(off, _CH)]], br, sem).wait()
        pltpu.make_async_copy(ns_hbm.at[idx_s_all.at[pl.ds(off, _CH)]], bs, sem).wait()
        pltpu.make_async_copy(p_hbm.at[pl.ds(base, _CH)], bp, psem).wait()

        @pl.when(c >= 2)
        def _():
            pltpu.make_async_copy(bo, out_hbm.at[pl.ds(w_base, _CH)], sst).wait()

        def row(r, acc):
            for cg in range(_D_OUT // 16):
                sl = pl.ds(cg * 16, 16)
                bo[r, sl] = jnp.maximum(bp[r, sl] + br[r, sl] + bs[r, sl], 0.0)
            return acc

        lax.fori_loop(0, _CH, row, 0)
        pltpu.async_copy(bo, out_hbm.at[pl.ds(base, _CH)], sst)

        if do_g:
            fire_g(c + 3, g)
        if do_p:
            fire_p(c + 2, q)

    fire_g(0, gsets[0])
    fire_g(1, gsets[1])
    fire_g(2, gsets[2])
    fire_p(0, qsets[0])
    fire_p(1, qsets[1])

    def body(j, carry):
        for t in range(6):
            step(6 * j + t, gsets[t % 3], qsets[t % 2], True, True)
        return carry

    lax.fori_loop(0, (_N_IT - 5) // 6, body, 0)
    for c in range(_N_IT - 5, _N_IT):
        step(c, gsets[c % 3], qsets[c % 2], c + 3 < _N_IT, c + 2 < _N_IT)
    pltpu.make_async_copy(bo_0, out_hbm.at[pl.ds(w_base, _CH)], sst_0).wait()
    pltpu.make_async_copy(bo_1, out_hbm.at[pl.ds(w_base, _CH)], sst_1).wait()


def kernel(nodes, edges, globals_attr, senders, receivers, W, b):
    we = W[:_D_EDGE]
    wr = W[_D_EDGE:_D_EDGE + _D_FEAT]
    ws = W[_D_EDGE + _D_FEAT:_D_EDGE + 2 * _D_FEAT]
    wg = W[_D_EDGE + 2 * _D_FEAT:]
    nr, ns = _make_tables(nodes, wr, ws)
    p = _make_p(edges.T, we, globals_attr, wg, b.reshape(1, _D_OUT))
    return _sc_edge(p, nr, ns, receivers, senders)
